# baseline (speedup 1.0000x reference)
import functools

import jax
import jax.numpy as jnp
from jax import lax
from jax.experimental import pallas as pl
from jax.experimental.pallas import tpu as pltpu
from jax.experimental.pallas import tpu_sc as plsc

_R = 2048
_K = 1024
_V = 32
_BR = 1024
_G = _R // _BR
_NC = 4
_DP = 128


def _exact_tree_d2(sq):
    a = sq[..., 0:4] + sq[..., 4:8]
    b = a[..., 0:2] + a[..., 2:4]
    c = b[..., 0] + b[..., 1]
    return c


def _tc_body(x_ref, emb_ref, idx_ref, d2_ref, hist_ref, ent_ref, tbl_ref):
    i = pl.program_id(0)
    x = x_ref[...]
    emb = emb_ref[...]

    e2 = jnp.sum(emb * emb, axis=1, keepdims=True)
    ea = jnp.concatenate([emb * -2.0, e2], axis=1)
    xa = jnp.concatenate([x, jnp.ones((_BR, 1), jnp.float32)], axis=1)
    s = lax.dot_general(
        xa, ea, (((1,), (1,)), ((), ())),
        precision=lax.Precision.HIGHEST,
        preferred_element_type=jnp.float32,
    )

    it = lax.broadcasted_iota(jnp.int32, (_BR, _K), 1)
    cand_idx = []
    for _ in range(_NC):
        m = jnp.min(s, axis=1, keepdims=True)
        ij = jnp.min(jnp.where(s == m, it, _K), axis=1, keepdims=True)
        cand_idx.append(ij)
        s = jnp.where(it == ij, jnp.inf, s)

    itk = lax.broadcasted_iota(jnp.int32, (_BR, _K), 1)
    cands = []
    for ij in cand_idx:
        oh = jnp.where(itk == ij, 1.0, 0.0).astype(jnp.float32)
        cands.append(lax.dot_general(
            oh, emb, (((1,), (0,)), ((), ())),
            precision=lax.Precision.HIGHEST,
            preferred_element_type=jnp.float32,
        ))
    cand_all = jnp.concatenate(cands, axis=1)
    xq = jnp.concatenate([x] * _NC, axis=1)
    df = xq - cand_all
    sq = (df * df).reshape(_BR, _NC, 4, 8)
    p = _exact_tree_d2(sq)
    d2c = ((p[..., 0] + p[..., 1]) + p[..., 2]) + p[..., 3]
    dc = d2c * lax.rsqrt(d2c)

    wd = dc[:, 0:1]
    wq = d2c[:, 0:1]
    wi = cand_idx[0]
    for j in range(1, _NC):
        dj = dc[:, j:j + 1]
        qj = d2c[:, j:j + 1]
        ij = cand_idx[j]
        better = (dj < wd) | ((dj == wd) & (ij < wi))
        wd = jnp.where(better, dj, wd)
        wq = jnp.where(better, qj, wq)
        wi = jnp.where(better, ij, wi)

    idx_ref[0, 0, :] = wi[:, 0]
    d2_ref[0, 0, :] = wq[:, 0]

    oh_w = jnp.where(it == wi, 1.0, 0.0).astype(jnp.float32)
    h = jnp.sum(oh_w, axis=0, keepdims=True)

    @pl.when(i == 0)
    def _init():
        hist_ref[...] = jnp.zeros_like(hist_ref)
        tbl_ref[:, 0:_V] = emb
        tbl_ref[:, _V:] = jnp.zeros((_K, _DP - _V), jnp.float32)

    hist_ref[...] += h

    @pl.when(i == _G - 1)
    def _fin():
        hh = hist_ref[...]
        prob = hh * (1.0 / _R)
        safe = jnp.where(hh > 0, prob, 1.0)
        ent_ref[...] = (-jnp.sum(safe * jnp.log(safe)))[None, None]


_tc_call = pl.pallas_call(
    _tc_body,
    grid=(_G,),
    in_specs=[
        pl.BlockSpec((_BR, _V), lambda i: (i, 0)),
        pl.BlockSpec((_K, _V), lambda i: (0, 0)),
    ],
    out_specs=[
        pl.BlockSpec((1, 1, _BR), lambda i: (i, 0, 0)),
        pl.BlockSpec((1, 1, _BR), lambda i: (i, 0, 0)),
        pl.BlockSpec((1, _K), lambda i: (0, 0)),
        pl.BlockSpec((1, 1), lambda i: (0, 0)),
        pl.BlockSpec((_K, _DP), lambda i: (0, 0)),
    ],
    out_shape=[
        jax.ShapeDtypeStruct((_G, 1, _BR), jnp.int32),
        jax.ShapeDtypeStruct((_G, 1, _BR), jnp.float32),
        jax.ShapeDtypeStruct((1, _K), jnp.float32),
        jax.ShapeDtypeStruct((1, 1), jnp.float32),
        jax.ShapeDtypeStruct((_K, _DP), jnp.float32),
    ],
)


@functools.cache
def _sc_gather():
    info = plsc.get_sparse_core_info()
    nc, ns = info.num_cores, info.num_subcores
    nw = nc * ns
    bw = _R // nw
    mesh = plsc.VectorSubcoreMesh(core_axis_name="c", subcore_axis_name="s")

    @functools.partial(
        pl.kernel,
        mesh=mesh,
        out_type=jax.ShapeDtypeStruct((_R, _DP), jnp.float32),
        scratch_types=[
            pltpu.VMEM((bw,), jnp.int32),
            pltpu.VMEM((bw, _DP), jnp.float32),
            pltpu.SemaphoreType.DMA,
        ],
    )
    def gather_k(table_hbm, idx_hbm, out_hbm, idx_v, rows_v, sem):
        wid = lax.axis_index("s") * nc + lax.axis_index("c")
        base = wid * bw
        pltpu.sync_copy(idx_hbm.at[pl.ds(base, bw)], idx_v)
        pltpu.async_copy(table_hbm.at[idx_v], rows_v, sem).wait()
        pltpu.sync_copy(rows_v, out_hbm.at[pl.ds(base, bw)])

    return gather_k


def kernel(x0, embedding0):
    x2 = x0.reshape(_R, _V)
    emb = embedding0.reshape(_K, _V)
    idx4, d24, _hist, ent, table = _tc_call(x2, emb)
    idx = idx4.reshape(_R)
    out0 = _sc_gather()(table, idx)[:, :_V].reshape(x0.shape)
    out1 = d24.reshape(x0.shape[0], x0.shape[1], x0.shape[2])
    return (out0, out1, out1, ent[0, 0])


# speedup vs baseline: 2.1265x; 2.1265x over previous
import functools

import jax
import jax.numpy as jnp
from jax import lax
from jax.experimental import pallas as pl
from jax.experimental.pallas import tpu as pltpu
from jax.experimental.pallas import tpu_sc as plsc

_R = 2048
_K = 1024
_V = 32
_BR = 1024
_G = _R // _BR
_NC = 4
_DP = 128


def _tc_body(x_ref, emb_ref, idx_ref, d2_ref, hist_ref, ent_ref, tbl_ref):
    i = pl.program_id(0)
    x = x_ref[...]
    emb = emb_ref[...]

    e2 = jnp.sum(emb * emb, axis=1, keepdims=True)
    ea = jnp.concatenate([emb * -2.0, e2], axis=1)
    xa = jnp.concatenate([x, jnp.ones((_BR, 1), jnp.float32)], axis=1)
    s = lax.dot_general(
        xa, ea, (((1,), (1,)), ((), ())),
        precision=lax.Precision.HIGHEST,
        preferred_element_type=jnp.float32,
    )

    it = lax.broadcasted_iota(jnp.int32, (_BR, _K), 1)
    cand_idx = []
    for _ in range(_NC):
        m = jnp.min(s, axis=1, keepdims=True)
        ij = jnp.min(jnp.where(s == m, it, _K), axis=1, keepdims=True)
        cand_idx.append(ij)
        s = jnp.where(it == ij, jnp.inf, s)

    ij_stack = jnp.concatenate(cand_idx, axis=0)
    its = lax.broadcasted_iota(jnp.int32, (_NC * _BR, _K), 1)
    ohs = jnp.where(its == ij_stack, 1.0, 0.0).astype(jnp.float32)
    cand_rows = lax.dot_general(
        ohs, emb, (((1,), (0,)), ((), ())),
        precision=lax.Precision.HIGHEST,
        preferred_element_type=jnp.float32,
    )
    cand_all = jnp.concatenate(
        [cand_rows[j * _BR:(j + 1) * _BR, :] for j in range(_NC)], axis=1)
    xq = jnp.concatenate([x] * _NC, axis=1)
    df = xq - cand_all
    sq = df * df

    a = sq + pltpu.roll(sq, 4, 1)
    b = a + pltpu.roll(a, 2, 1)
    c = b + pltpu.roll(b, 1, 1)
    e1 = c + pltpu.roll(c, 8, 1)
    g1 = pltpu.roll(e1, 8, 1) + c
    g2 = pltpu.roll(g1, 8, 1) + c
    d2cols = [g2[:, 32 * j + 31:32 * j + 32] for j in range(_NC)]
    dcols = [q * lax.rsqrt(q) for q in d2cols]

    wd = dcols[0]
    wq = d2cols[0]
    wi = cand_idx[0]
    for j in range(1, _NC):
        dj = dcols[j]
        qj = d2cols[j]
        ij = cand_idx[j]
        better = (dj < wd) | ((dj == wd) & (ij < wi))
        wd = jnp.where(better, dj, wd)
        wq = jnp.where(better, qj, wq)
        wi = jnp.where(better, ij, wi)

    idx_ref[0, 0, :] = wi[:, 0]
    d2_ref[0, 0, :] = wq[:, 0]

    oh_w = jnp.where(it == wi, 1.0, 0.0).astype(jnp.float32)
    h = jnp.sum(oh_w, axis=0, keepdims=True)

    @pl.when(i == 0)
    def _init():
        hist_ref[...] = jnp.zeros_like(hist_ref)
        tbl_ref[:, 0:_V] = emb
        tbl_ref[:, _V:] = jnp.zeros((_K, _DP - _V), jnp.float32)

    hist_ref[...] += h

    @pl.when(i == _G - 1)
    def _fin():
        hh = hist_ref[...]
        prob = hh * (1.0 / _R)
        safe = jnp.where(hh > 0, prob, 1.0)
        ent_ref[...] = (-jnp.sum(safe * jnp.log(safe)))[None, None]


_tc_call = pl.pallas_call(
    _tc_body,
    grid=(_G,),
    in_specs=[
        pl.BlockSpec((_BR, _V), lambda i: (i, 0)),
        pl.BlockSpec((_K, _V), lambda i: (0, 0)),
    ],
    out_specs=[
        pl.BlockSpec((1, 1, _BR), lambda i: (i, 0, 0)),
        pl.BlockSpec((1, 1, _BR), lambda i: (i, 0, 0)),
        pl.BlockSpec((1, _K), lambda i: (0, 0)),
        pl.BlockSpec((1, 1), lambda i: (0, 0)),
        pl.BlockSpec((_K, _DP), lambda i: (0, 0)),
    ],
    out_shape=[
        jax.ShapeDtypeStruct((_G, 1, _BR), jnp.int32),
        jax.ShapeDtypeStruct((_G, 1, _BR), jnp.float32),
        jax.ShapeDtypeStruct((1, _K), jnp.float32),
        jax.ShapeDtypeStruct((1, 1), jnp.float32),
        jax.ShapeDtypeStruct((_K, _DP), jnp.float32),
    ],
)


@functools.cache
def _sc_gather():
    info = plsc.get_sparse_core_info()
    nc, ns = info.num_cores, info.num_subcores
    nw = nc * ns
    bw = _R // nw
    mesh = plsc.VectorSubcoreMesh(core_axis_name="c", subcore_axis_name="s")

    @functools.partial(
        pl.kernel,
        mesh=mesh,
        out_type=jax.ShapeDtypeStruct((_R, _DP), jnp.float32),
        scratch_types=[
            pltpu.VMEM((bw,), jnp.int32),
            pltpu.VMEM((bw, _DP), jnp.float32),
            pltpu.SemaphoreType.DMA,
        ],
    )
    def gather_k(table_hbm, idx_hbm, out_hbm, idx_v, rows_v, sem):
        wid = lax.axis_index("s") * nc + lax.axis_index("c")
        base = wid * bw
        pltpu.sync_copy(idx_hbm.at[pl.ds(base, bw)], idx_v)
        pltpu.async_copy(table_hbm.at[idx_v], rows_v, sem).wait()
        pltpu.sync_copy(rows_v, out_hbm.at[pl.ds(base, bw)])

    return gather_k


def kernel(x0, embedding0):
    x2 = x0.reshape(_R, _V)
    emb = embedding0.reshape(_K, _V)
    idx4, d24, _hist, ent, table = _tc_call(x2, emb)
    idx = idx4.reshape(_R)
    out0 = _sc_gather()(table, idx)[:, :_V].reshape(x0.shape)
    out1 = d24.reshape(x0.shape[0], x0.shape[1], x0.shape[2])
    return (out0, out1, out1, ent[0, 0])


# speedup vs baseline: 2.6778x; 1.2593x over previous
import functools

import jax
import jax.numpy as jnp
from jax import lax
from jax.experimental import pallas as pl
from jax.experimental.pallas import tpu as pltpu
from jax.experimental.pallas import tpu_sc as plsc

_R = 2048
_K = 1024
_V = 32
_BR = 512
_G = _R // _BR
_DP = 128

_GROUPS = [[8 * j + s for s in range(8)] for j in range(4)]


def _fold8(g):
    return ((g[0] + g[4]) + (g[2] + g[6])) + ((g[1] + g[5]) + (g[3] + g[7]))


def _tc_body(x_ref, emb_ref, idx_ref, d2_ref, hist_ref, ent_ref, tbl_ref):
    i = pl.program_id(0)
    x = x_ref[...]
    emb = emb_ref[...]
    et = jnp.transpose(emb)
    acc = None
    for grp in _GROUPS:
        planes = []
        for v in grp:
            dv = x[:, v:v + 1] - et[v:v + 1, :]
            planes.append(dv * dv)
        p = _fold8(planes)
        acc = p if acc is None else acc + p
    d = acc * lax.rsqrt(acc)
    iota = lax.broadcasted_iota(jnp.int32, (_BR, _K), 1)
    dmin = jnp.min(d, axis=1, keepdims=True)
    idx = jnp.min(jnp.where(d == dmin, iota, _K), axis=1)
    idx_ref[...] = idx
    d2_ref[0, 0, :] = jnp.min(acc, axis=1)
    oh = jnp.where(iota == idx[:, None], 1.0, 0.0).astype(jnp.float32)
    h = jnp.sum(oh, axis=0, keepdims=True)

    @pl.when(i == 0)
    def _init():
        hist_ref[...] = jnp.zeros_like(hist_ref)
        tbl_ref[:, 0:_V] = emb
        tbl_ref[:, _V:] = jnp.zeros((_K, _DP - _V), jnp.float32)

    hist_ref[...] += h

    @pl.when(i == _G - 1)
    def _fin():
        hh = hist_ref[...]
        prob = hh * (1.0 / _R)
        safe = jnp.where(hh > 0, prob, 1.0)
        ent_ref[...] = (-jnp.sum(safe * jnp.log(safe)))[None, None]


_tc_call = pl.pallas_call(
    _tc_body,
    grid=(_G,),
    in_specs=[
        pl.BlockSpec((_BR, _V), lambda i: (i, 0)),
        pl.BlockSpec((_K, _V), lambda i: (0, 0)),
    ],
    out_specs=[
        pl.BlockSpec((_BR,), lambda i: (i,)),
        pl.BlockSpec((1, 1, _BR), lambda i: (i, 0, 0)),
        pl.BlockSpec((1, _K), lambda i: (0, 0)),
        pl.BlockSpec((1, 1), lambda i: (0, 0)),
        pl.BlockSpec((_K, _DP), lambda i: (0, 0)),
    ],
    out_shape=[
        jax.ShapeDtypeStruct((_R,), jnp.int32),
        jax.ShapeDtypeStruct((_G, 1, _BR), jnp.float32),
        jax.ShapeDtypeStruct((1, _K), jnp.float32),
        jax.ShapeDtypeStruct((1, 1), jnp.float32),
        jax.ShapeDtypeStruct((_K, _DP), jnp.float32),
    ],
)


@functools.cache
def _sc_gather():
    info = plsc.get_sparse_core_info()
    nc, ns = info.num_cores, info.num_subcores
    nw = nc * ns
    bw = _R // nw
    mesh = plsc.VectorSubcoreMesh(core_axis_name="c", subcore_axis_name="s")

    @functools.partial(
        pl.kernel,
        mesh=mesh,
        out_type=jax.ShapeDtypeStruct((_R, _DP), jnp.float32),
        scratch_types=[
            pltpu.VMEM((bw,), jnp.int32),
            pltpu.VMEM((bw, _DP), jnp.float32),
            pltpu.SemaphoreType.DMA,
        ],
    )
    def gather_k(table_hbm, idx_hbm, out_hbm, idx_v, rows_v, sem):
        wid = lax.axis_index("s") * nc + lax.axis_index("c")
        base = wid * bw
        pltpu.sync_copy(idx_hbm.at[pl.ds(base, bw)], idx_v)
        pltpu.async_copy(table_hbm.at[idx_v], rows_v, sem).wait()
        pltpu.sync_copy(rows_v, out_hbm.at[pl.ds(base, bw)])

    return gather_k


def kernel(x0, embedding0):
    x2 = x0.reshape(_R, _V)
    emb = embedding0.reshape(_K, _V)
    idx, d24, _hist, ent, table = _tc_call(x2, emb)
    out0 = _sc_gather()(table, idx)[:, :_V].reshape(x0.shape)
    out1 = d24.reshape(x0.shape[0], x0.shape[1], x0.shape[2])
    return (out0, out1, out1, ent[0, 0])


# speedup vs baseline: 2.7149x; 1.0138x over previous
import functools

import jax
import jax.numpy as jnp
from jax import lax
from jax.experimental import pallas as pl
from jax.experimental.pallas import tpu as pltpu
from jax.experimental.pallas import tpu_sc as plsc

_R = 2048
_K = 1024
_V = 32
_BR = 512
_G = _R // _BR

_GROUPS = [[8 * j + s for s in range(8)] for j in range(4)]


def _fold8(g):
    return ((g[0] + g[4]) + (g[2] + g[6])) + ((g[1] + g[5]) + (g[3] + g[7]))


def _tc_body(x_ref, et_ref, idx_ref, d2_ref, hist_ref, ent_ref):
    i = pl.program_id(0)
    x = x_ref[...]
    et = et_ref[...]
    acc = None
    for grp in _GROUPS:
        planes = []
        for v in grp:
            dv = x[:, v:v + 1] - et[v:v + 1, :]
            planes.append(dv * dv)
        p = _fold8(planes)
        acc = p if acc is None else acc + p
    d = acc * lax.rsqrt(acc)
    iota = lax.broadcasted_iota(jnp.int32, (_BR, _K), 1)
    dmin = jnp.min(d, axis=1, keepdims=True)
    idx = jnp.min(jnp.where(d == dmin, iota, _K), axis=1)
    idx_ref[0, 0, :] = idx
    d2_ref[0, 0, :] = jnp.min(acc, axis=1)
    oh = jnp.where(iota == idx[:, None], 1.0, 0.0).astype(jnp.float32)
    h = jnp.sum(oh, axis=0, keepdims=True)

    @pl.when(i == 0)
    def _init():
        hist_ref[...] = jnp.zeros_like(hist_ref)

    hist_ref[...] += h

    @pl.when(i == _G - 1)
    def _fin():
        hh = hist_ref[...]
        prob = hh * (1.0 / _R)
        safe = jnp.where(hh > 0, prob, 1.0)
        ent_ref[...] = (-jnp.sum(safe * jnp.log(safe)))[None, None]


_tc_call = pl.pallas_call(
    _tc_body,
    grid=(_G,),
    in_specs=[
        pl.BlockSpec((_BR, _V), lambda i: (i, 0)),
        pl.BlockSpec((_V, _K), lambda i: (0, 0)),
    ],
    out_specs=[
        pl.BlockSpec((1, 1, _BR), lambda i: (i, 0, 0)),
        pl.BlockSpec((1, 1, _BR), lambda i: (i, 0, 0)),
        pl.BlockSpec((1, _K), lambda i: (0, 0)),
        pl.BlockSpec((1, 1), lambda i: (0, 0)),
    ],
    out_shape=[
        jax.ShapeDtypeStruct((_G, 1, _BR), jnp.int32),
        jax.ShapeDtypeStruct((_G, 1, _BR), jnp.float32),
        jax.ShapeDtypeStruct((1, _K), jnp.float32),
        jax.ShapeDtypeStruct((1, 1), jnp.float32),
    ],
)


_DP = 128


@functools.cache
def _sc_gather():
    info = plsc.get_sparse_core_info()
    nc, ns = info.num_cores, info.num_subcores
    nw = nc * ns
    bw = _R // nw
    mesh = plsc.VectorSubcoreMesh(core_axis_name="c", subcore_axis_name="s")

    @functools.partial(
        pl.kernel,
        mesh=mesh,
        out_type=jax.ShapeDtypeStruct((_R, _DP), jnp.float32),
        scratch_types=[
            pltpu.VMEM((bw,), jnp.int32),
            pltpu.VMEM((bw, _DP), jnp.float32),
            pltpu.SemaphoreType.DMA,
        ],
    )
    def gather_k(table_hbm, idx_hbm, out_hbm, idx_v, rows_v, sem):
        wid = lax.axis_index("s") * nc + lax.axis_index("c")
        base = wid * bw
        pltpu.sync_copy(idx_hbm.at[pl.ds(base, bw)], idx_v)
        pltpu.async_copy(table_hbm.at[idx_v], rows_v, sem).wait()
        pltpu.sync_copy(rows_v, out_hbm.at[pl.ds(base, bw)])

    return gather_k


def kernel(x0, embedding0):
    x2 = x0.reshape(_R, _V)
    emb = embedding0.reshape(_K, _V)
    idx4, d24, _hist, ent = _tc_call(x2, emb.T)
    idx = idx4.reshape(_R)
    table = jnp.pad(emb, ((0, 0), (0, _DP - _V)))
    out0 = _sc_gather()(table, idx)[:, :_V].reshape(x0.shape)
    out1 = d24.reshape(x0.shape[0], x0.shape[1], x0.shape[2])
    return (out0, out1, out1, ent[0, 0])
